# Initial kernel scaffold; baseline (speedup 1.0000x reference)
import jax
import jax.numpy as jnp
from jax.experimental import pallas as pl


def kernel(x, edge_index, edge_attr, u, batch, g1, be1, W1, c1, g2, be2, W2, c2, g3, be3, W3, c3):
    raise NotImplementedError("write your pallas kernel here")


# speedup vs baseline: 3.0371x; 3.0371x over previous
import functools

import jax
import jax.numpy as jnp
from jax import lax
from jax.experimental import pallas as pl
from jax.experimental.pallas import tpu as pltpu
from jax.experimental.pallas import tpu_sc as plsc

_N = 10000
_D = 128
_B = 64
_EPS = 1e-5
_LEAK = 0.0

_NC = 2
_NS = 16
_NW = _NC * _NS
_CHUNK = 320
_LAST_ROWS = _N - (_NW - 1) * _CHUNK


def _sc_segment_sums(x, batch):
    mesh = plsc.VectorSubcoreMesh(core_axis_name="c", subcore_axis_name="s")

    @functools.partial(
        pl.kernel,
        mesh=mesh,
        compiler_params=pltpu.CompilerParams(needs_layout_passes=False),
        out_type=[
            jax.ShapeDtypeStruct((_NW, _B * _D), jnp.float32),
            jax.ShapeDtypeStruct((_NW, _B * 16), jnp.float32),
        ],
        scratch_types=[
            pltpu.VMEM((_CHUNK, _D), jnp.float32),
            pltpu.VMEM((_CHUNK,), jnp.int32),
            pltpu.VMEM((_B * _D,), jnp.float32),
            pltpu.VMEM((_B * 16,), jnp.float32),
        ],
    )
    def seg_kernel(x_hbm, b_hbm, out_sum, out_cnt, x_v, b_v, acc, cnt):
        cid = lax.axis_index("c")
        sid = lax.axis_index("s")
        wid = cid * _NS + sid
        base = wid * _CHUNK

        iota16 = lax.iota(jnp.int32, 16)
        zeros16 = jnp.zeros((16,), jnp.float32)
        one0 = jnp.where(iota16 == 0, 1.0, 0.0).astype(jnp.float32)

        def _zsum(i, _):
            acc[pl.ds(i * 16, 16)] = zeros16
            return _
        lax.fori_loop(0, _B * _D // 16, _zsum, 0)

        def _zcnt(i, _):
            cnt[pl.ds(i * 16, 16)] = zeros16
            return _
        lax.fori_loop(0, _B, _zcnt, 0)

        @pl.when(wid < _NW - 1)
        def _():
            pltpu.sync_copy(x_hbm.at[pl.ds(base, _CHUNK)], x_v)
            pltpu.sync_copy(b_hbm.at[pl.ds(base, _CHUNK)], b_v)

        @pl.when(wid == _NW - 1)
        def _():
            pltpu.sync_copy(x_hbm.at[pl.ds(base, _LAST_ROWS)],
                            x_v.at[pl.ds(0, _LAST_ROWS)])
            pltpu.sync_copy(b_hbm.at[pl.ds(base, _LAST_ROWS)],
                            b_v.at[pl.ds(0, _LAST_ROWS)])

        groups = jnp.where(wid == _NW - 1, _LAST_ROWS // 16, _CHUNK // 16)

        col_idx = [iota16 + (j * 16) for j in range(_D // 16)]

        def _group(g, _):
            segs = b_v[pl.ds(g * 16, 16)]
            for i in range(16):
                r = g * 16 + i
                sum_base = jnp.full((16,), segs[i] * _D, jnp.int32)
                for j in range(_D // 16):
                    v = x_v[r, pl.ds(j * 16, 16)]
                    plsc.addupdate_scatter(acc, [sum_base + col_idx[j]], v)
                cnt_idx = jnp.full((16,), segs[i] * 16, jnp.int32) + iota16
                plsc.addupdate_scatter(cnt, [cnt_idx], one0)
            return _
        lax.fori_loop(0, groups, _group, 0)

        pltpu.sync_copy(acc, out_sum.at[wid])
        pltpu.sync_copy(cnt, out_cnt.at[wid])

    psum, pcnt = seg_kernel(x, batch)
    return psum.reshape(_NW, _B, _D), pcnt.reshape(_NW, _B, 16)


def _tc_mlp(psum, pcnt, u, g1, be1, W1, c1, g2, be2, W2, c2, g3, be3, W3, c3):
    def body(ps, pc, u_r, g1_r, be1_r, W1_r, c1_r, g2_r, be2_r, W2_r, c2_r,
             g3_r, be3_r, W3_r, c3_r, out):
        s = jnp.sum(ps[...], axis=0)
        cnt16 = jnp.sum(pc[...], axis=0)
        cnt = cnt16[:, 0:1]
        pooled = s / jnp.clip(cnt, 1.0)
        h = jnp.concatenate([u_r[...], pooled], axis=1)

        def bn(h, g, b):
            mu = jnp.mean(h, axis=0, keepdims=True)
            var = jnp.mean((h - mu) * (h - mu), axis=0, keepdims=True)
            return g * (h - mu) * lax.rsqrt(var + _EPS) + b

        def lrelu(h):
            return jnp.where(h >= 0, h, _LEAK * h)

        h = bn(h, g1_r[...], be1_r[...])
        h = lrelu(jnp.dot(h, W1_r[...], preferred_element_type=jnp.float32)
                  + c1_r[...])
        h = bn(h, g2_r[...], be2_r[...])
        h = lrelu(jnp.dot(h, W2_r[...], preferred_element_type=jnp.float32)
                  + c2_r[...])
        h = bn(h, g3_r[...], be3_r[...])
        out[...] = (jnp.dot(h, W3_r[...], preferred_element_type=jnp.float32)
                    + c3_r[...])

    return pl.pallas_call(
        body,
        out_shape=jax.ShapeDtypeStruct((_B, W3.shape[1]), jnp.float32),
    )(psum, pcnt, u, g1.reshape(1, -1), be1.reshape(1, -1), W1,
      c1.reshape(1, -1), g2.reshape(1, -1), be2.reshape(1, -1), W2,
      c2.reshape(1, -1), g3.reshape(1, -1), be3.reshape(1, -1), W3,
      c3.reshape(1, -1))


def kernel(x, edge_index, edge_attr, u, batch,
           g1, be1, W1, c1, g2, be2, W2, c2, g3, be3, W3, c3):
    del edge_index, edge_attr
    psum, pcnt = _sc_segment_sums(x, batch)
    return _tc_mlp(psum, pcnt, u, g1, be1, W1, c1,
                   g2, be2, W2, c2, g3, be3, W3, c3)


# speedup vs baseline: 4.2827x; 1.4101x over previous
import functools

import jax
import jax.numpy as jnp
from jax import lax
from jax.experimental import pallas as pl
from jax.experimental.pallas import tpu as pltpu
from jax.experimental.pallas import tpu_sc as plsc

_N = 10000
_D = 128
_B = 64
_EPS = 1e-5
_LEAK = 0.0

_NC = 2
_NS = 16
_NW = _NC * _NS
_CHUNK = 320
_LAST_ROWS = _N - (_NW - 1) * _CHUNK
_NJ = _D // 16


def _sc_segment_sums(x, batch):
    mesh = plsc.VectorSubcoreMesh(core_axis_name="c", subcore_axis_name="s")

    @functools.partial(
        pl.kernel,
        mesh=mesh,
        compiler_params=pltpu.CompilerParams(needs_layout_passes=False),
        out_type=[
            jax.ShapeDtypeStruct((_NW, _B, _D), jnp.float32),
            jax.ShapeDtypeStruct((_NW, _B, 16), jnp.float32),
        ],
        scratch_types=[
            pltpu.VMEM((_CHUNK, _D), jnp.float32),
            pltpu.VMEM((_CHUNK,), jnp.int32),
            pltpu.VMEM((_B, _D), jnp.float32),
            pltpu.VMEM((_B, 16), jnp.float32),
            pltpu.SemaphoreType.DMA,
            pltpu.SemaphoreType.DMA,
        ],
    )
    def seg_kernel(x_hbm, b_hbm, out_sum, out_cnt, x_v, b_v, acc, cnt,
                   sem_x, sem_b):
        cid = lax.axis_index("c")
        sid = lax.axis_index("s")
        wid = cid * _NS + sid
        base = wid * _CHUNK

        iota16 = lax.iota(jnp.int32, 16)
        zeros16 = jnp.zeros((16,), jnp.float32)

        last = wid == _NW - 1

        @pl.when(jnp.logical_not(last))
        def _():
            pltpu.async_copy(x_hbm.at[pl.ds(base, _CHUNK)], x_v, sem_x)
            pltpu.async_copy(b_hbm.at[pl.ds(base, _CHUNK)], b_v, sem_b)

        @pl.when(last)
        def _():
            pltpu.async_copy(x_hbm.at[pl.ds(base, _LAST_ROWS)],
                             x_v.at[pl.ds(0, _LAST_ROWS)], sem_x)
            pltpu.async_copy(b_hbm.at[pl.ds(base, _LAST_ROWS)],
                             b_v.at[pl.ds(0, _LAST_ROWS)], sem_b)

        for r in range(_B):
            for j in range(_NJ):
                acc[r, pl.ds(j * 16, 16)] = zeros16
            cnt[r, :] = zeros16

        @pl.when(jnp.logical_not(last))
        def _():
            pltpu.make_async_copy(x_hbm.at[pl.ds(base, _CHUNK)], x_v,
                                  sem_x).wait()
            pltpu.make_async_copy(b_hbm.at[pl.ds(base, _CHUNK)], b_v,
                                  sem_b).wait()

        @pl.when(last)
        def _():
            pltpu.make_async_copy(x_hbm.at[pl.ds(base, _LAST_ROWS)],
                                  x_v.at[pl.ds(0, _LAST_ROWS)], sem_x).wait()
            pltpu.make_async_copy(b_hbm.at[pl.ds(base, _LAST_ROWS)],
                                  b_v.at[pl.ds(0, _LAST_ROWS)], sem_b).wait()

        groups = jnp.where(last, _LAST_ROWS // 16, _CHUNK // 16)

        first = b_v[pl.ds(0, 16)][0]

        def _flush(seg, cnt_run, sums):
            for j in range(_NJ):
                acc[seg, pl.ds(j * 16, 16)] = sums[j]
            cnt[seg, :] = jnp.where(iota16 == 0, cnt_run, 0.0)

        def _group(g, carry):
            cur_seg, cnt_run, *sums = carry
            segs = b_v[pl.ds(g * 16, 16)]
            for i in range(16):
                r = g * 16 + i
                seg_i = segs[i]
                changed = seg_i != cur_seg

                @pl.when(changed)
                def _(cur_seg=cur_seg, cnt_run=cnt_run, sums=tuple(sums)):
                    _flush(cur_seg, cnt_run, sums)

                new_sums = []
                for j in range(_NJ):
                    v = x_v[r, pl.ds(j * 16, 16)]
                    new_sums.append(jnp.where(changed, v, sums[j] + v))
                sums = new_sums
                cnt_run = jnp.where(changed, 1.0, cnt_run + 1.0)
                cur_seg = seg_i
            return (cur_seg, cnt_run, *sums)

        init = (first, jnp.float32(0.0)) + tuple(zeros16 for _ in range(_NJ))
        final = lax.fori_loop(0, groups, _group, init)
        _flush(final[0], final[1], final[2:])

        pltpu.sync_copy(acc, out_sum.at[wid])
        pltpu.sync_copy(cnt, out_cnt.at[wid])

    return seg_kernel(x, batch)


def _tc_mlp(psum, pcnt, u, g1, be1, W1, c1, g2, be2, W2, c2, g3, be3, W3, c3):
    def body(ps, pc, u_r, g1_r, be1_r, W1_r, c1_r, g2_r, be2_r, W2_r, c2_r,
             g3_r, be3_r, W3_r, c3_r, out):
        s = jnp.sum(ps[...], axis=0)
        cnt16 = jnp.sum(pc[...], axis=0)
        cnt = cnt16[:, 0:1]
        pooled = s / jnp.clip(cnt, 1.0)
        h = jnp.concatenate([u_r[...], pooled], axis=1)

        def bn(h, g, b):
            mu = jnp.mean(h, axis=0, keepdims=True)
            var = jnp.mean((h - mu) * (h - mu), axis=0, keepdims=True)
            return g * (h - mu) * lax.rsqrt(var + _EPS) + b

        def lrelu(h):
            return jnp.where(h >= 0, h, _LEAK * h)

        h = bn(h, g1_r[...], be1_r[...])
        h = lrelu(jnp.dot(h, W1_r[...], preferred_element_type=jnp.float32)
                  + c1_r[...])
        h = bn(h, g2_r[...], be2_r[...])
        h = lrelu(jnp.dot(h, W2_r[...], preferred_element_type=jnp.float32)
                  + c2_r[...])
        h = bn(h, g3_r[...], be3_r[...])
        out[...] = (jnp.dot(h, W3_r[...], preferred_element_type=jnp.float32)
                    + c3_r[...])

    return pl.pallas_call(
        body,
        out_shape=jax.ShapeDtypeStruct((_B, W3.shape[1]), jnp.float32),
    )(psum, pcnt, u, g1.reshape(1, -1), be1.reshape(1, -1), W1,
      c1.reshape(1, -1), g2.reshape(1, -1), be2.reshape(1, -1), W2,
      c2.reshape(1, -1), g3.reshape(1, -1), be3.reshape(1, -1), W3,
      c3.reshape(1, -1))


def kernel(x, edge_index, edge_attr, u, batch,
           g1, be1, W1, c1, g2, be2, W2, c2, g3, be3, W3, c3):
    del edge_index, edge_attr
    psum, pcnt = _sc_segment_sums(x, batch)
    return _tc_mlp(psum, pcnt, u, g1, be1, W1, c1,
                   g2, be2, W2, c2, g3, be3, W3, c3)


# speedup vs baseline: 4.7780x; 1.1156x over previous
import functools

import jax
import jax.numpy as jnp
from jax import lax
from jax.experimental import pallas as pl
from jax.experimental.pallas import tpu as pltpu
from jax.experimental.pallas import tpu_sc as plsc

_N = 10000
_D = 128
_B = 64
_EPS = 1e-5
_LEAK = 0.0

_NC = 2
_NS = 16
_NW = _NC * _NS
_SUB = 80
_NSUB = 4
_CHUNK = _SUB * _NSUB
_LAST_SUBS = (_N - (_NW - 1) * _CHUNK) // _SUB


def _sc_segment_sums(x, batch):
    mesh = plsc.VectorSubcoreMesh(core_axis_name="c", subcore_axis_name="s")

    @functools.partial(
        pl.kernel,
        mesh=mesh,
        compiler_params=pltpu.CompilerParams(needs_layout_passes=False),
        out_type=jax.ShapeDtypeStruct((_NC, _B, _D), jnp.float32),
        scratch_types=[
            pltpu.VMEM((_SUB, _D), jnp.float32),
            pltpu.VMEM((_SUB, _D), jnp.float32),
            pltpu.VMEM((_NSUB, _SUB), jnp.int32),
            pltpu.VMEM((_B, _D), jnp.float32),
            pltpu.VMEM_SHARED((_B, _D), jnp.float32),
            pltpu.SemaphoreType.DMA,
            pltpu.SemaphoreType.DMA,
            pltpu.SemaphoreType.DMA,
        ],
    )
    def seg_kernel(x_hbm, b_hbm, out_sum, xb0, xb1, b_v, zv, sh_sum,
                   sem0, sem1, sem_b):
        cid = lax.axis_index("c")
        sid = lax.axis_index("s")
        wid = cid * _NS + sid
        xbase = wid * _CHUNK
        last = wid == _NW - 1
        zeros16 = jnp.zeros((16,), jnp.float32)

        @pl.when(jnp.logical_not(last))
        def _():
            for g in range(_NSUB):
                pltpu.async_copy(b_hbm.at[pl.ds(xbase + g * _SUB, _SUB)],
                                 b_v.at[g], sem_b)

        @pl.when(last)
        def _():
            pltpu.async_copy(b_hbm.at[pl.ds(xbase, _SUB)], b_v.at[0], sem_b)

        pltpu.async_copy(x_hbm.at[pl.ds(xbase, _SUB)], xb0, sem0)

        @pl.when(sid == 0)
        def _():
            def _zrow(r, c):
                for j in range(_D // 16):
                    zv[r, pl.ds(j * 16, 16)] = zeros16
                return c
            lax.fori_loop(0, _B, _zrow, 0)
            pltpu.sync_copy(zv, sh_sum)

        plsc.subcore_barrier()

        @pl.when(jnp.logical_not(last))
        def _():
            for g in range(_NSUB):
                pltpu.make_async_copy(b_hbm.at[pl.ds(xbase + g * _SUB, _SUB)],
                                      b_v.at[g], sem_b).wait()
            for g in range(_NSUB):
                cur, sc = (xb0, sem0) if g % 2 == 0 else (xb1, sem1)
                nxt, sn = (xb1, sem1) if g % 2 == 0 else (xb0, sem0)
                pltpu.make_async_copy(
                    x_hbm.at[pl.ds(xbase + g * _SUB, _SUB)], cur, sc).wait()
                if g < _NSUB - 1:
                    pltpu.async_copy(
                        x_hbm.at[pl.ds(xbase + (g + 1) * _SUB, _SUB)],
                        nxt, sn)
                pltpu.sync_copy(cur, sh_sum.at[b_v.at[g]], add=True)

        @pl.when(last)
        def _():
            pltpu.make_async_copy(b_hbm.at[pl.ds(xbase, _SUB)], b_v.at[0],
                                  sem_b).wait()
            pltpu.make_async_copy(x_hbm.at[pl.ds(xbase, _SUB)], xb0,
                                  sem0).wait()
            pltpu.sync_copy(xb0, sh_sum.at[b_v.at[0]], add=True)

        plsc.subcore_barrier()

        @pl.when(sid == 0)
        def _():
            pltpu.sync_copy(sh_sum, out_sum.at[cid])

    return seg_kernel(x, batch)


def _tc_mlp(psum, batch_row, u, g1, be1, W1, c1, g2, be2, W2, c2,
            g3, be3, W3, c3):
    def body(ps, b_r, u_r, g1_r, be1_r, W1_r, c1_r, g2_r, be2_r, W2_r, c2_r,
             g3_r, be3_r, W3_r, c3_r, out):
        s = ps[0] + ps[1]
        seg_ids = lax.broadcasted_iota(jnp.int32, (_B, 1), 0)
        onehot = (b_r[...] == seg_ids).astype(jnp.float32)
        ones_col = jnp.ones((_N, 1), jnp.float32)
        cnt = jnp.dot(onehot, ones_col,
                      preferred_element_type=jnp.float32)
        pooled = s / jnp.clip(cnt, 1.0)
        h = jnp.concatenate([u_r[...], pooled], axis=1)

        def bn(h, g, b):
            mu = jnp.mean(h, axis=0, keepdims=True)
            var = jnp.mean((h - mu) * (h - mu), axis=0, keepdims=True)
            return g * (h - mu) * lax.rsqrt(var + _EPS) + b

        def lrelu(h):
            return jnp.where(h >= 0, h, _LEAK * h)

        h = bn(h, g1_r[...], be1_r[...])
        h = lrelu(jnp.dot(h, W1_r[...], preferred_element_type=jnp.float32)
                  + c1_r[...])
        h = bn(h, g2_r[...], be2_r[...])
        h = lrelu(jnp.dot(h, W2_r[...], preferred_element_type=jnp.float32)
                  + c2_r[...])
        h = bn(h, g3_r[...], be3_r[...])
        out[...] = (jnp.dot(h, W3_r[...], preferred_element_type=jnp.float32)
                    + c3_r[...])

    return pl.pallas_call(
        body,
        out_shape=jax.ShapeDtypeStruct((_B, W3.shape[1]), jnp.float32),
    )(psum, batch_row, u, g1.reshape(1, -1), be1.reshape(1, -1), W1,
      c1.reshape(1, -1), g2.reshape(1, -1), be2.reshape(1, -1), W2,
      c2.reshape(1, -1), g3.reshape(1, -1), be3.reshape(1, -1), W3,
      c3.reshape(1, -1))


def kernel(x, edge_index, edge_attr, u, batch,
           g1, be1, W1, c1, g2, be2, W2, c2, g3, be3, W3, c3):
    del edge_index, edge_attr
    psum = _sc_segment_sums(x, batch)
    return _tc_mlp(psum, batch.reshape(1, _N), u, g1, be1, W1, c1,
                   g2, be2, W2, c2, g3, be3, W3, c3)


# speedup vs baseline: 4.9927x; 1.0449x over previous
import functools

import jax
import jax.numpy as jnp
from jax import lax
from jax.experimental import pallas as pl
from jax.experimental.pallas import tpu as pltpu
from jax.experimental.pallas import tpu_sc as plsc

_N = 10000
_D = 128
_B = 64
_EPS = 1e-5
_LEAK = 0.0

_NC = 2
_NS = 16
_NW = _NC * _NS
_SUB = 80
_NSUB = 4
_CHUNK = _SUB * _NSUB
_LAST_SUBS = (_N - (_NW - 1) * _CHUNK) // _SUB


def _sc_segment_sums(x, batch):
    mesh = plsc.VectorSubcoreMesh(core_axis_name="c", subcore_axis_name="s")

    @functools.partial(
        pl.kernel,
        mesh=mesh,
        compiler_params=pltpu.CompilerParams(needs_layout_passes=False),
        out_type=jax.ShapeDtypeStruct((_NC, _B, _D), jnp.float32),
        scratch_types=[
            pltpu.VMEM((_NSUB, _SUB, _D), jnp.float32),
            pltpu.VMEM((_NSUB, _SUB), jnp.int32),
            pltpu.VMEM((_B, _D), jnp.float32),
            pltpu.VMEM_SHARED((_B, _D), jnp.float32),
            pltpu.SemaphoreType.DMA,
            pltpu.SemaphoreType.DMA,
            pltpu.SemaphoreType.DMA,
        ],
    )
    def seg_kernel(x_hbm, b_hbm, out_sum, xb, b_v, zv, sh_sum,
                   sem_st, sem_sc, sem_b):
        cid = lax.axis_index("c")
        sid = lax.axis_index("s")
        wid = cid * _NS + sid
        xbase = wid * _CHUNK
        last = wid == _NW - 1
        nsub = jnp.where(last, _LAST_SUBS, _NSUB)
        zeros16 = jnp.zeros((16,), jnp.float32)

        pltpu.async_copy(b_hbm.at[pl.ds(xbase, _SUB)], b_v.at[0], sem_b)
        pltpu.async_copy(x_hbm.at[pl.ds(xbase, _SUB)], xb.at[0], sem_st)

        @pl.when(jnp.logical_not(last))
        def _():
            for g in range(1, _NSUB):
                pltpu.async_copy(b_hbm.at[pl.ds(xbase + g * _SUB, _SUB)],
                                 b_v.at[g], sem_b)
                pltpu.async_copy(x_hbm.at[pl.ds(xbase + g * _SUB, _SUB)],
                                 xb.at[g], sem_st)

        @pl.when(sid == 0)
        def _():
            def _zrow(r, c):
                for j in range(_D // 16):
                    zv[r, pl.ds(j * 16, 16)] = zeros16
                return c
            lax.fori_loop(0, _B, _zrow, 0)
            pltpu.sync_copy(zv, sh_sum)

        plsc.subcore_barrier()

        def _run(g):
            pltpu.make_async_copy(
                x_hbm.at[pl.ds(xbase + g * _SUB, _SUB)], xb.at[g],
                sem_st).wait()
            pltpu.make_async_copy(
                b_hbm.at[pl.ds(xbase + g * _SUB, _SUB)], b_v.at[g],
                sem_b).wait()
            pltpu.async_copy(xb.at[g], sh_sum.at[b_v.at[g]], sem_sc,
                             add=True)

        _run(0)

        @pl.when(jnp.logical_not(last))
        def _():
            for g in range(1, _NSUB):
                _run(g)

        def _drain(g):
            pltpu.make_async_copy(xb.at[g], sh_sum.at[b_v.at[g]],
                                  sem_sc).wait()

        _drain(0)

        @pl.when(jnp.logical_not(last))
        def _():
            for g in range(1, _NSUB):
                _drain(g)

        plsc.subcore_barrier()

        @pl.when(sid == 0)
        def _():
            pltpu.sync_copy(sh_sum, out_sum.at[cid])

    return seg_kernel(x, batch)


def _tc_mlp(psum, batch, u, g1, be1, W1, c1, g2, be2, W2, c2,
            g3, be3, W3, c3):
    def body(ps, b_r, u_r, g1_r, be1_r, W1_r, c1_r, g2_r, be2_r, W2_r, c2_r,
             g3_r, be3_r, W3_r, c3_r, out):
        s = ps[0] + ps[1]
        seg_ids = lax.broadcasted_iota(jnp.int32, (_B, 1), 0)
        b_row = b_r[...][None, :]
        onehot = (b_row == seg_ids).astype(jnp.float32)
        ones_col = jnp.ones((_N, 1), jnp.float32)
        cnt = jnp.dot(onehot, ones_col,
                      preferred_element_type=jnp.float32)
        pooled = s / jnp.clip(cnt, 1.0)
        h = jnp.concatenate([u_r[...], pooled], axis=1)

        def bn(h, g_v, b_v):
            mu = jnp.mean(h, axis=0, keepdims=True)
            var = jnp.mean((h - mu) * (h - mu), axis=0, keepdims=True)
            return (g_v[...][None, :] * (h - mu) * lax.rsqrt(var + _EPS)
                    + b_v[...][None, :])

        def lrelu(h):
            return jnp.where(h >= 0, h, _LEAK * h)

        h = bn(h, g1_r, be1_r)
        h = lrelu(jnp.dot(h, W1_r[...], preferred_element_type=jnp.float32)
                  + c1_r[...][None, :])
        h = bn(h, g2_r, be2_r)
        h = lrelu(jnp.dot(h, W2_r[...], preferred_element_type=jnp.float32)
                  + c2_r[...][None, :])
        h = bn(h, g3_r, be3_r)
        out[...] = (jnp.dot(h, W3_r[...], preferred_element_type=jnp.float32)
                    + c3_r[...][None, :])

    return pl.pallas_call(
        body,
        out_shape=jax.ShapeDtypeStruct((_B, W3.shape[1]), jnp.float32),
    )(psum, batch, u, g1, be1, W1, c1, g2, be2, W2, c2, g3, be3, W3, c3)


def kernel(x, edge_index, edge_attr, u, batch,
           g1, be1, W1, c1, g2, be2, W2, c2, g3, be3, W3, c3):
    del edge_index, edge_attr
    psum = _sc_segment_sums(x, batch)
    return _tc_mlp(psum, batch, u, g1, be1, W1, c1,
                   g2, be2, W2, c2, g3, be3, W3, c3)


# speedup vs baseline: 5.0027x; 1.0020x over previous
import functools

import jax
import jax.numpy as jnp
from jax import lax
from jax.experimental import pallas as pl
from jax.experimental.pallas import tpu as pltpu
from jax.experimental.pallas import tpu_sc as plsc

_N = 10000
_D = 128
_B = 64
_EPS = 1e-5
_LEAK = 0.0

_NC = 2
_NS = 16
_NW = _NC * _NS
_SUB = 80
_NSUB = 4
_CHUNK = _SUB * _NSUB
_LAST_SUBS = (_N - (_NW - 1) * _CHUNK) // _SUB


def _sc_segment_sums(x, batch):
    mesh = plsc.VectorSubcoreMesh(core_axis_name="c", subcore_axis_name="s")

    @functools.partial(
        pl.kernel,
        mesh=mesh,
        compiler_params=pltpu.CompilerParams(needs_layout_passes=False,
                                             skip_device_barrier=True),
        out_type=jax.ShapeDtypeStruct((_NC, _B, _D), jnp.float32),
        scratch_types=[
            pltpu.VMEM((_NSUB, _SUB, _D), jnp.float32),
            pltpu.VMEM((_NSUB, _SUB), jnp.int32),
            pltpu.VMEM((_B, _D), jnp.float32),
            pltpu.VMEM_SHARED((_B, _D), jnp.float32),
            pltpu.SemaphoreType.DMA,
            pltpu.SemaphoreType.DMA,
            pltpu.SemaphoreType.DMA,
        ],
    )
    def seg_kernel(x_hbm, b_hbm, out_sum, xb, b_v, zv, sh_sum,
                   sem_st, sem_sc, sem_b):
        cid = lax.axis_index("c")
        sid = lax.axis_index("s")
        wid = cid * _NS + sid
        xbase = wid * _CHUNK
        last = wid == _NW - 1
        nsub = jnp.where(last, _LAST_SUBS, _NSUB)
        zeros16 = jnp.zeros((16,), jnp.float32)

        pltpu.async_copy(b_hbm.at[pl.ds(xbase, _SUB)], b_v.at[0], sem_b)
        pltpu.async_copy(x_hbm.at[pl.ds(xbase, _SUB)], xb.at[0], sem_st)

        @pl.when(jnp.logical_not(last))
        def _():
            for g in range(1, _NSUB):
                pltpu.async_copy(b_hbm.at[pl.ds(xbase + g * _SUB, _SUB)],
                                 b_v.at[g], sem_b)
                pltpu.async_copy(x_hbm.at[pl.ds(xbase + g * _SUB, _SUB)],
                                 xb.at[g], sem_st)

        @pl.when(sid == 0)
        def _():
            def _zrow(r, c):
                for j in range(_D // 16):
                    zv[r, pl.ds(j * 16, 16)] = zeros16
                return c
            lax.fori_loop(0, _B, _zrow, 0)
            pltpu.sync_copy(zv, sh_sum)

        plsc.subcore_barrier()

        def _run(g):
            pltpu.make_async_copy(
                x_hbm.at[pl.ds(xbase + g * _SUB, _SUB)], xb.at[g],
                sem_st).wait()
            pltpu.make_async_copy(
                b_hbm.at[pl.ds(xbase + g * _SUB, _SUB)], b_v.at[g],
                sem_b).wait()
            pltpu.async_copy(xb.at[g], sh_sum.at[b_v.at[g]], sem_sc,
                             add=True)

        _run(0)

        @pl.when(jnp.logical_not(last))
        def _():
            for g in range(1, _NSUB):
                _run(g)

        def _drain(g):
            pltpu.make_async_copy(xb.at[g], sh_sum.at[b_v.at[g]],
                                  sem_sc).wait()

        _drain(0)

        @pl.when(jnp.logical_not(last))
        def _():
            for g in range(1, _NSUB):
                _drain(g)

        plsc.subcore_barrier()

        @pl.when(sid == 0)
        def _():
            pltpu.sync_copy(sh_sum, out_sum.at[cid])

    return seg_kernel(x, batch)


def _tc_mlp(psum, batch, u, g1, be1, W1, c1, g2, be2, W2, c2,
            g3, be3, W3, c3):
    def body(ps, b_r, u_r, g1_r, be1_r, W1_r, c1_r, g2_r, be2_r, W2_r, c2_r,
             g3_r, be3_r, W3_r, c3_r, out):
        s = ps[0] + ps[1]
        seg_ids = lax.broadcasted_iota(jnp.int32, (_B, 1), 0)
        b_row = b_r[...][None, :]
        onehot = (b_row == seg_ids).astype(jnp.float32)
        ones_col = jnp.ones((_N, 1), jnp.float32)
        cnt = jnp.dot(onehot, ones_col,
                      preferred_element_type=jnp.float32)
        pooled = s / jnp.clip(cnt, 1.0)
        h = jnp.concatenate([u_r[...], pooled], axis=1)

        def bn(h, g_v, b_v):
            mu = jnp.mean(h, axis=0, keepdims=True)
            var = jnp.mean((h - mu) * (h - mu), axis=0, keepdims=True)
            return (g_v[...][None, :] * (h - mu) * lax.rsqrt(var + _EPS)
                    + b_v[...][None, :])

        def lrelu(h):
            return jnp.where(h >= 0, h, _LEAK * h)

        h = bn(h, g1_r, be1_r)
        h = lrelu(jnp.dot(h, W1_r[...], preferred_element_type=jnp.float32)
                  + c1_r[...][None, :])
        h = bn(h, g2_r, be2_r)
        h = lrelu(jnp.dot(h, W2_r[...], preferred_element_type=jnp.float32)
                  + c2_r[...][None, :])
        h = bn(h, g3_r, be3_r)
        out[...] = (jnp.dot(h, W3_r[...], preferred_element_type=jnp.float32)
                    + c3_r[...][None, :])

    return pl.pallas_call(
        body,
        out_shape=jax.ShapeDtypeStruct((_B, W3.shape[1]), jnp.float32),
    )(psum, batch, u, g1, be1, W1, c1, g2, be2, W2, c2, g3, be3, W3, c3)


def kernel(x, edge_index, edge_attr, u, batch,
           g1, be1, W1, c1, g2, be2, W2, c2, g3, be3, W3, c3):
    del edge_index, edge_attr
    psum = _sc_segment_sums(x, batch)
    return _tc_mlp(psum, batch, u, g1, be1, W1, c1,
                   g2, be2, W2, c2, g3, be3, W3, c3)
